# baseline (speedup 1.0000x reference)
import functools

import jax
import jax.numpy as jnp
from jax import lax
from jax.experimental import pallas as pl

B, C, H, W = 64, 384, 32, 32
HW = H * W
E = 8
RED = 48
BB = 8


def _body(x_ref, w1_ref, b1_ref, w2_ref, b2_ref, out_ref):
    xs = x_ref[...]
    pooled = jnp.mean(xs, axis=2)
    h = jnp.dot(pooled, w1_ref[...], preferred_element_type=jnp.float32)
    h = h + b1_ref[...]
    h = h * jax.nn.sigmoid(h)
    logits = jnp.dot(h, w2_ref[...], preferred_element_type=jnp.float32)
    logits = logits + b2_ref[...]
    w = jax.nn.softmax(logits, axis=1)
    idx = lax.broadcasted_iota(jnp.int32, (BB, E), 1)
    m1 = jnp.max(w, axis=1, keepdims=True)
    i1 = jnp.min(jnp.where(w == m1, idx, E), axis=1, keepdims=True)
    w_rest = jnp.where(idx == i1, -jnp.inf, w)
    m2 = jnp.max(w_rest, axis=1, keepdims=True)
    i2 = jnp.min(jnp.where(w_rest == m2, idx, E), axis=1, keepdims=True)
    mask = (idx == i1) | (idx == i2)
    wsel = jnp.where(mask, w, 0.0)
    wn = wsel / (jnp.sum(wsel, axis=1, keepdims=True) + 1e-8)
    out_ref[...] = jnp.broadcast_to(wn[:, :, None], (BB, E, HW))


@jax.jit
def kernel(x, W1, b1, W2, b2):
    xr = x.reshape(B, C, HW)
    out = pl.pallas_call(
        _body,
        grid=(B // BB,),
        in_specs=[
            pl.BlockSpec((BB, C, HW), lambda i: (i, 0, 0)),
            pl.BlockSpec((C, RED), lambda i: (0, 0)),
            pl.BlockSpec((1, RED), lambda i: (0, 0)),
            pl.BlockSpec((RED, E), lambda i: (0, 0)),
            pl.BlockSpec((1, E), lambda i: (0, 0)),
        ],
        out_specs=pl.BlockSpec((BB, E, HW), lambda i: (i, 0, 0)),
        out_shape=jax.ShapeDtypeStruct((B, E, HW), jnp.float32),
    )(xr, W1, b1.reshape(1, RED), W2, b2.reshape(1, E))
    return out.reshape(B, E, H, W)
